# baseline (speedup 1.0000x reference)
import functools

import jax
import jax.numpy as jnp
from jax import lax
from jax.experimental import pallas as pl
from jax.experimental.pallas import tpu as pltpu
from jax.experimental.pallas import tpu_sc as plsc

B = 4096
S = 200
D = 64
NW = 32
RPW = B // NW
NCHUNK = 13
WPAD = 224
NEG = -1e30
UNROLL = 8
NBUF = 4

_mesh = plsc.VectorSubcoreMesh(core_axis_name="c", subcore_axis_name="s")

_GDN = lax.GatherDimensionNumbers(
    offset_dims=(), collapsed_slice_dims=(0,), start_index_map=(0,))


def _lane_shuffle(v, idx):
    return lax.gather(v, idx[:, None], _GDN, (1,),
                      mode=lax.GatherScatterMode.PROMISE_IN_BOUNDS)


def _butterfly(v, op):
    lane = lax.iota(jnp.int32, 16)
    for d in (1, 2, 4, 8):
        v = op(v, _lane_shuffle(v, lane ^ d))
    return v


_scratch = [pltpu.VMEM((RPW, S), jnp.int32)]
_scratch += [pltpu.VMEM((WPAD,), jnp.float32)] * NBUF
_scratch += [pltpu.VMEM((S, D), jnp.float32)] * NBUF
_scratch += [pltpu.VMEM((WPAD,), jnp.float32)]
_scratch += [pltpu.VMEM((RPW, D), jnp.float32)]
_scratch += [pltpu.SemaphoreType.DMA] * NBUF


@functools.partial(
    pl.kernel,
    out_type=jax.ShapeDtypeStruct((B, D), jnp.float32),
    mesh=_mesh,
    compiler_params=pltpu.CompilerParams(
        needs_layout_passes=False, use_tc_tiling_on_sc=False),
    scratch_types=_scratch,
)
def _encode(doc_hbm, emb_hbm, wt_hbm, out_hbm, doc_v, *bufs):
    w_bufs = bufs[:NBUF]
    e_bufs = bufs[NBUF:2 * NBUF]
    p_v = bufs[2 * NBUF]
    o_v = bufs[2 * NBUF + 1]
    sems = bufs[2 * NBUF + 2:]

    cid = lax.axis_index("c")
    sid = lax.axis_index("s")
    wid = sid * 2 + cid
    base = wid * RPW

    pltpu.sync_copy(doc_hbm.at[pl.ds(base, RPW)], doc_v)

    for w_v in w_bufs:
        w_v[pl.ds(200, 16)] = jnp.full((16,), NEG, jnp.float32)

    _SPLITS = ((0, 56), (56, 48), (104, 48), (152, 48))

    def copies(r, e_v, w_v, sem):
        out = []
        for (o, n) in _SPLITS:
            i = doc_v.at[r, pl.ds(o, n)]
            out.append(pltpu.make_async_copy(
                emb_hbm.at[i], e_v.at[pl.ds(o, n)], sem))
        for (o, n) in ((0, 104), (104, 96)):
            i = doc_v.at[r, pl.ds(o, n)]
            out.append(pltpu.make_async_copy(
                wt_hbm.at[i], w_v.at[pl.ds(o, n)], sem))
        return tuple(out)

    def issue(r, e_v, w_v, sem):
        @pl.when(r < RPW)
        def _():
            for c in copies(r, e_v, w_v, sem):
                c.start()

    def drain(r, e_v, w_v, sem):
        for c in copies(r, e_v, w_v, sem):
            c.wait()

    def compute(r, e_v, w_v):
        def max_body(k, m):
            return jnp.maximum(m, w_v[pl.ds(k * 16, 16)])

        mvec = lax.fori_loop(0, NCHUNK, max_body,
                             jnp.full((16,), NEG, jnp.float32))
        m = _butterfly(mvec, jnp.maximum)

        def exp_body(k, s):
            p = jnp.exp(w_v[pl.ds(k * 16, 16)] - m)
            p_v[pl.ds(k * 16, 16)] = p
            return s + p

        svec = lax.fori_loop(0, NCHUNK, exp_body,
                             jnp.zeros((16,), jnp.float32))
        inv = 1.0 / _butterfly(svec, jnp.add)

        def tok_body(tb, accs):
            a0, a1, a2, a3 = accs
            t0 = tb * UNROLL
            for j in range(UNROLL):
                t = t0 + j
                pb = plsc.load_gather(p_v, [jnp.full((16,), t, jnp.int32)])
                a0 = a0 + pb * e_v[t, pl.ds(0, 16)]
                a1 = a1 + pb * e_v[t, pl.ds(16, 16)]
                a2 = a2 + pb * e_v[t, pl.ds(32, 16)]
                a3 = a3 + pb * e_v[t, pl.ds(48, 16)]
            return (a0, a1, a2, a3)

        z = jnp.zeros((16,), jnp.float32)
        a0, a1, a2, a3 = lax.fori_loop(0, S // UNROLL, tok_body, (z, z, z, z))
        o_v[r, pl.ds(0, 16)] = a0 * inv
        o_v[r, pl.ds(16, 16)] = a1 * inv
        o_v[r, pl.ds(32, 16)] = a2 * inv
        o_v[r, pl.ds(48, 16)] = a3 * inv

    for k in range(NBUF):
        issue(k, e_bufs[k], w_bufs[k], sems[k])

    def group_body(i, carry):
        r0 = i * NBUF
        for k in range(NBUF):
            r = r0 + k
            drain(r, e_bufs[k], w_bufs[k], sems[k])
            compute(r, e_bufs[k], w_bufs[k])
            issue(r + NBUF, e_bufs[k], w_bufs[k], sems[k])
        return carry

    lax.fori_loop(0, RPW // NBUF, group_body, 0)
    pltpu.sync_copy(o_v, out_hbm.at[pl.ds(base, RPW)])


@jax.jit
def kernel(document, lens, embed_table, weight_table):
    del lens
    doc = document.astype(jnp.int32)
    wt = weight_table.reshape(-1)
    return _encode(doc, embed_table, wt)


# speedup vs baseline: 1.0022x; 1.0022x over previous
import functools

import jax
import jax.numpy as jnp
from jax import lax
from jax.experimental import pallas as pl
from jax.experimental.pallas import tpu as pltpu
from jax.experimental.pallas import tpu_sc as plsc

B = 4096
S = 200
D = 64
NW = 32
RPW = B // NW
NCHUNK = 13
WPAD = 224
NEG = -1e30
UNROLL = 8
NBUF = 4

_mesh = plsc.VectorSubcoreMesh(core_axis_name="c", subcore_axis_name="s")

_GDN = lax.GatherDimensionNumbers(
    offset_dims=(), collapsed_slice_dims=(0,), start_index_map=(0,))


def _lane_shuffle(v, idx):
    return lax.gather(v, idx[:, None], _GDN, (1,),
                      mode=lax.GatherScatterMode.PROMISE_IN_BOUNDS)


def _butterfly(v, op):
    lane = lax.iota(jnp.int32, 16)
    for d in (1, 2, 4, 8):
        v = op(v, _lane_shuffle(v, lane ^ d))
    return v


_scratch = [pltpu.VMEM((RPW, S), jnp.int32)]
_scratch += [pltpu.VMEM((WPAD,), jnp.float32)] * NBUF
_scratch += [pltpu.VMEM((S, D), jnp.float32)] * NBUF
_scratch += [pltpu.VMEM((WPAD,), jnp.float32)]
_scratch += [pltpu.VMEM((RPW, D), jnp.float32)]
_scratch += [pltpu.SemaphoreType.DMA] * NBUF


@functools.partial(
    pl.kernel,
    out_type=jax.ShapeDtypeStruct((B, D), jnp.float32),
    mesh=_mesh,
    compiler_params=pltpu.CompilerParams(
        needs_layout_passes=False, use_tc_tiling_on_sc=False),
    scratch_types=_scratch,
)
def _encode(doc_hbm, emb_hbm, wt_hbm, out_hbm, doc_v, *bufs):
    w_bufs = bufs[:NBUF]
    e_bufs = bufs[NBUF:2 * NBUF]
    p_v = bufs[2 * NBUF]
    o_v = bufs[2 * NBUF + 1]
    sems = bufs[2 * NBUF + 2:]

    cid = lax.axis_index("c")
    sid = lax.axis_index("s")
    wid = sid * 2 + cid
    base = wid * RPW

    pltpu.sync_copy(doc_hbm.at[pl.ds(base, RPW)], doc_v)

    for w_v in w_bufs:
        w_v[pl.ds(200, 16)] = jnp.full((16,), NEG, jnp.float32)

    _SPLITS = ((0, 200),)

    def copies(r, e_v, w_v, sem):
        out = []
        for (o, n) in _SPLITS:
            i = doc_v.at[r, pl.ds(o, n)]
            out.append(pltpu.make_async_copy(
                emb_hbm.at[i], e_v.at[pl.ds(o, n)], sem))
        for (o, n) in ((0, 104), (104, 96)):
            i = doc_v.at[r, pl.ds(o, n)]
            out.append(pltpu.make_async_copy(
                wt_hbm.at[i], w_v.at[pl.ds(o, n)], sem))
        return tuple(out)

    def issue(r, e_v, w_v, sem):
        @pl.when(r < RPW)
        def _():
            for c in copies(r, e_v, w_v, sem):
                c.start()

    def drain(r, e_v, w_v, sem):
        for c in copies(r, e_v, w_v, sem):
            c.wait()

    def compute(r, e_v, w_v):
        def max_body(k, m):
            return jnp.maximum(m, w_v[pl.ds(k * 16, 16)])

        mvec = lax.fori_loop(0, NCHUNK, max_body,
                             jnp.full((16,), NEG, jnp.float32))
        m = _butterfly(mvec, jnp.maximum)

        def exp_body(k, s):
            p = jnp.exp(w_v[pl.ds(k * 16, 16)] - m)
            p_v[pl.ds(k * 16, 16)] = p
            return s + p

        svec = lax.fori_loop(0, NCHUNK, exp_body,
                             jnp.zeros((16,), jnp.float32))
        inv = 1.0 / _butterfly(svec, jnp.add)

        def tok_body(tb, accs):
            a0, a1, a2, a3 = accs
            t0 = tb * UNROLL
            for j in range(UNROLL):
                t = t0 + j
                pb = plsc.load_gather(p_v, [jnp.full((16,), t, jnp.int32)])
                a0 = a0 + pb * e_v[t, pl.ds(0, 16)]
                a1 = a1 + pb * e_v[t, pl.ds(16, 16)]
                a2 = a2 + pb * e_v[t, pl.ds(32, 16)]
                a3 = a3 + pb * e_v[t, pl.ds(48, 16)]
            return (a0, a1, a2, a3)

        z = jnp.zeros((16,), jnp.float32)
        a0, a1, a2, a3 = lax.fori_loop(0, S // UNROLL, tok_body, (z, z, z, z))
        o_v[r, pl.ds(0, 16)] = a0 * inv
        o_v[r, pl.ds(16, 16)] = a1 * inv
        o_v[r, pl.ds(32, 16)] = a2 * inv
        o_v[r, pl.ds(48, 16)] = a3 * inv

    for k in range(NBUF):
        issue(k, e_bufs[k], w_bufs[k], sems[k])

    def group_body(i, carry):
        r0 = i * NBUF
        for k in range(NBUF):
            r = r0 + k
            drain(r, e_bufs[k], w_bufs[k], sems[k])
            compute(r, e_bufs[k], w_bufs[k])
            issue(r + NBUF, e_bufs[k], w_bufs[k], sems[k])
        return carry

    lax.fori_loop(0, RPW // NBUF, group_body, 0)
    pltpu.sync_copy(o_v, out_hbm.at[pl.ds(base, RPW)])


@jax.jit
def kernel(document, lens, embed_table, weight_table):
    del lens
    doc = document.astype(jnp.int32)
    wt = weight_table.reshape(-1)
    return _encode(doc, embed_table, wt)
